# baseline (speedup 1.0000x reference)
import numpy as np
import jax
import jax.numpy as jnp
from jax.experimental import pallas as pl

_FINE = 0.7
_MEDIAN = 0.4

_ROWS = 1024 * 64 * 64 // 128
_BLOCK_ROWS = 1024

_P_NP = np.zeros((128, 384), dtype=np.float32)
for _j in range(128):
    for _k in range(3):
        _P_NP[_j, 3 * _j + _k] = 1.0
_P_BF16 = jnp.asarray(_P_NP, dtype=jnp.bfloat16)

_KPAT = jnp.asarray(np.arange(384) % 3, dtype=jnp.bfloat16)


def _gate_kernel(e_ref, p_ref, kpat_ref, out_ref):
    e = e_ref[...]
    cls = ((e > _MEDIAN).astype(jnp.bfloat16)
           + (e > _FINE).astype(jnp.bfloat16))
    rep = jax.lax.dot_general(
        cls, p_ref[...],
        dimension_numbers=(((1,), (0,)), ((), ())),
        preferred_element_type=jnp.float32,
    )
    kpat = kpat_ref[...].astype(jnp.float32)
    out_ref[...] = (rep == kpat).astype(jnp.int32)


def kernel(entropy):
    e2 = entropy.reshape(_ROWS, 128)
    grid = _ROWS // _BLOCK_ROWS
    out = pl.pallas_call(
        _gate_kernel,
        grid=(grid,),
        in_specs=[
            pl.BlockSpec((_BLOCK_ROWS, 128), lambda i: (i, 0)),
            pl.BlockSpec((128, 384), lambda i: (0, 0)),
            pl.BlockSpec((1, 384), lambda i: (0, 0)),
        ],
        out_specs=pl.BlockSpec((_BLOCK_ROWS, 384), lambda i: (i, 0)),
        out_shape=jax.ShapeDtypeStruct((_ROWS, 384), jnp.int32),
    )(e2, _P_BF16, _KPAT.reshape(1, 384))
    return out.reshape(1024, 64, 64, 3)


# speedup vs baseline: 120.9811x; 120.9811x over previous
import jax
import jax.numpy as jnp
from jax.experimental import pallas as pl

_FINE = 0.7
_MEDIAN = 0.4

_BLK = 8


def _gate_kernel(e_ref, out_ref):
    e = e_ref[...]
    m_gt_med = e > _MEDIAN
    m_gt_fine = e > _FINE
    one = jnp.ones(e.shape, jnp.int32)
    zero = jnp.zeros(e.shape, jnp.int32)
    out_ref[:, 0, :, :] = jnp.where(m_gt_med, zero, one)
    out_ref[:, 1, :, :] = jnp.where(m_gt_med & (~m_gt_fine), one, zero)
    out_ref[:, 2, :, :] = jnp.where(m_gt_fine, one, zero)


def kernel(entropy):
    t = jnp.transpose(entropy, (1, 2, 0))
    grid = 64 // _BLK
    out = pl.pallas_call(
        _gate_kernel,
        grid=(grid,),
        in_specs=[pl.BlockSpec((_BLK, 64, 1024), lambda i: (i, 0, 0))],
        out_specs=pl.BlockSpec((_BLK, 3, 64, 1024), lambda i: (i, 0, 0, 0)),
        out_shape=jax.ShapeDtypeStruct((64, 3, 64, 1024), jnp.int32),
    )(t)
    return jnp.transpose(out, (3, 0, 2, 1))
